# Initial kernel scaffold; baseline (speedup 1.0000x reference)
import jax
import jax.numpy as jnp
from jax.experimental import pallas as pl


def kernel(node_embs, edge_embs, edge_list, node_list, W1, W2, a1, a2):
    raise NotImplementedError("write your pallas kernel here")


# speedup vs baseline: 3.0460x; 3.0460x over previous
import functools

import jax
import jax.numpy as jnp
from jax import lax
from jax.experimental import pallas as pl
from jax.experimental.pallas import tpu as pltpu
from jax.experimental.pallas import tpu_sc as plsc

N = 8192
M = 16384
A = 8
L = 16
D = 256
ALPHA = 0.2
NEG = -9e15

NC, NS = 2, 16
NW = NC * NS
LANES = 16


def _tc1_body(x_ref, w1_ref, a1_ref, a2lo_ref, wh_ref, sn_ref, a2hi_ref):
    x = x_ref[...]
    wh = jnp.dot(x, w1_ref[...], preferred_element_type=jnp.float32)
    wh_ref[...] = wh
    lrelu = jnp.where(wh > 0, wh, ALPHA * wh)
    sn_ref[...] = jnp.dot(lrelu, a1_ref[...], preferred_element_type=jnp.float32)
    a2hi_ref[...] = jnp.dot(x, a2lo_ref[...], preferred_element_type=jnp.float32)


def _tc_stage1(node_embs, W1, a1, a2lo):
    BLK = 1024
    return pl.pallas_call(
        _tc1_body,
        grid=(N // BLK,),
        in_specs=[
            pl.BlockSpec((BLK, D), lambda i: (i, 0)),
            pl.BlockSpec((D, D), lambda i: (0, 0)),
            pl.BlockSpec((D, 1), lambda i: (0, 0)),
            pl.BlockSpec((D, 1), lambda i: (0, 0)),
        ],
        out_specs=[
            pl.BlockSpec((BLK, D), lambda i: (i, 0)),
            pl.BlockSpec((BLK, 1), lambda i: (i, 0)),
            pl.BlockSpec((BLK, 1), lambda i: (i, 0)),
        ],
        out_shape=[
            jax.ShapeDtypeStruct((N, D), jnp.float32),
            jax.ShapeDtypeStruct((N, 1), jnp.float32),
            jax.ShapeDtypeStruct((N, 1), jnp.float32),
        ],
    )(node_embs, W1, a1, a2lo)


def _tc3_body(x_ref, w2_ref, a2c_ref, wf_ref, se_ref, eo_ref):
    x = x_ref[...]
    wf = jnp.dot(x, w2_ref[...], preferred_element_type=jnp.float32)
    wf_ref[...] = wf
    se_ref[...] = jnp.dot(wf, a2c_ref[...], preferred_element_type=jnp.float32)
    eo_ref[...] = jnp.where(x > 0, x, jnp.exp(x) - 1.0)


def _tc_stage3(new_edge, W2, a2c):
    BLK = 1024
    return pl.pallas_call(
        _tc3_body,
        grid=(M // BLK,),
        in_specs=[
            pl.BlockSpec((BLK, D), lambda i: (i, 0)),
            pl.BlockSpec((D, D), lambda i: (0, 0)),
            pl.BlockSpec((D, 1), lambda i: (0, 0)),
        ],
        out_specs=[
            pl.BlockSpec((BLK, D), lambda i: (i, 0)),
            pl.BlockSpec((BLK, 1), lambda i: (i, 0)),
            pl.BlockSpec((BLK, D), lambda i: (i, 0)),
        ],
        out_shape=[
            jax.ShapeDtypeStruct((M, D), jnp.float32),
            jax.ShapeDtypeStruct((M, 1), jnp.float32),
            jax.ShapeDtypeStruct((M, D), jnp.float32),
        ],
    )(new_edge, W2, a2c)


_MESH = plsc.VectorSubcoreMesh(
    core_axis_name="c", subcore_axis_name="s", num_cores=NC, num_subcores=NS)

E_CH = 16
E_PW = M // NW
N_CH = 16
N_PW = N // NW


def _softmax_weights(scores):
    mx = scores[0]
    for s in scores[1:]:
        mx = jnp.maximum(mx, s)
    es = [jnp.exp(s - mx) for s in scores]
    tot = es[0]
    for e in es[1:]:
        tot = tot + e
    inv = 1.0 / tot
    return [e * inv for e in es]


def _edge_agg_body(wh_hbm, sn_hbm, el_hbm, out_hbm,
                   sn_l, idx_l, rows_l, out_l, w_buf, sem):
    wid = lax.axis_index("s") * NC + lax.axis_index("c")
    pltpu.sync_copy(sn_hbm, sn_l)
    iota = lax.iota(jnp.int32, LANES)

    def chunk(c, _):
        base = wid * E_PW + c * E_CH
        pltpu.sync_copy(el_hbm.at[pl.ds(base * A, E_CH * A)], idx_l)
        gat = pltpu.async_copy(wh_hbm.at[idx_l], rows_l, sem)
        scores = []
        for a in range(A):
            nid = plsc.load_gather(idx_l, [iota * A + a])
            s = plsc.load_gather(sn_l, [nid])
            scores.append(jnp.where(s > 0, s, NEG))
        ws = _softmax_weights(scores)
        for a in range(A):
            w_buf[pl.ds(a * LANES, LANES)] = ws[a]
        gat.wait()

        def edge(e, _):
            ev = jnp.full((LANES,), e, dtype=jnp.int32)
            wv = [plsc.load_gather(w_buf, [ev + (a * LANES)]) for a in range(A)]
            rb = e * A
            for dc in range(D // LANES):
                sl = pl.ds(dc * LANES, LANES)
                acc = wv[0] * rows_l[rb, sl]
                for a in range(1, A):
                    acc = acc + wv[a] * rows_l[rb + a, sl]
                out_l[e, sl] = acc
            return 0

        lax.fori_loop(0, E_CH, edge, 0)
        pltpu.sync_copy(out_l, out_hbm.at[pl.ds(base, E_CH)])
        return 0

    lax.fori_loop(0, E_PW // E_CH, chunk, 0)


def _sc_edge_agg(Wh, sn, el_flat):
    return pl.kernel(
        _edge_agg_body,
        out_type=jax.ShapeDtypeStruct((M, D), jnp.float32),
        mesh=_MESH,
        scratch_types=[
            pltpu.VMEM((N,), jnp.float32),
            pltpu.VMEM((E_CH * A,), jnp.int32),
            pltpu.VMEM((E_CH * A, D), jnp.float32),
            pltpu.VMEM((E_CH, D), jnp.float32),
            pltpu.VMEM((A * LANES,), jnp.float32),
            pltpu.SemaphoreType.DMA,
        ],
        compiler_params=pltpu.CompilerParams(needs_layout_passes=False),
    )(Wh, sn, el_flat)


def _node_agg_body(wf_hbm, se_hbm, nl_hbm, a2hi_hbm, out_hbm,
                   se_l, idx_l, rows_l, out_l, w_buf, a2h_l, sem):
    wid = lax.axis_index("s") * NC + lax.axis_index("c")
    pltpu.sync_copy(se_hbm, se_l)
    iota = lax.iota(jnp.int32, LANES)

    def chunk(c, _):
        base = wid * N_PW + c * N_CH
        pltpu.sync_copy(nl_hbm.at[pl.ds(base * L, N_CH * L)], idx_l)
        pltpu.sync_copy(a2hi_hbm.at[pl.ds(base, N_CH)], a2h_l)
        gat = pltpu.async_copy(wf_hbm.at[idx_l], rows_l, sem)
        a2h = a2h_l[...]
        scores = []
        for l in range(L):
            eid = plsc.load_gather(idx_l, [iota * L + l])
            s = plsc.load_gather(se_l, [eid])
            scores.append(jnp.where(s != 0, s + a2h, s))
        ws = _softmax_weights(scores)
        for l in range(L):
            w_buf[pl.ds(l * LANES, LANES)] = ws[l]
        gat.wait()

        def node(n, _):
            nv = jnp.full((LANES,), n, dtype=jnp.int32)
            wv = [plsc.load_gather(w_buf, [nv + (l * LANES)]) for l in range(L)]
            rb = n * L
            for dc in range(D // LANES):
                sl = pl.ds(dc * LANES, LANES)
                acc = wv[0] * rows_l[rb, sl]
                for l in range(1, L):
                    acc = acc + wv[l] * rows_l[rb + l, sl]
                out_l[n, sl] = jnp.where(acc > 0, acc, jnp.exp(acc) - 1.0)
            return 0

        lax.fori_loop(0, N_CH, node, 0)
        pltpu.sync_copy(out_l, out_hbm.at[pl.ds(base, N_CH)])
        return 0

    lax.fori_loop(0, N_PW // N_CH, chunk, 0)


def _sc_node_agg(Wf, se, nl_flat, a2hi):
    return pl.kernel(
        _node_agg_body,
        out_type=jax.ShapeDtypeStruct((N, D), jnp.float32),
        mesh=_MESH,
        scratch_types=[
            pltpu.VMEM((M,), jnp.float32),
            pltpu.VMEM((N_CH * L,), jnp.int32),
            pltpu.VMEM((N_CH * L, D), jnp.float32),
            pltpu.VMEM((N_CH, D), jnp.float32),
            pltpu.VMEM((L * LANES,), jnp.float32),
            pltpu.VMEM((N_CH,), jnp.float32),
            pltpu.SemaphoreType.DMA,
        ],
        compiler_params=pltpu.CompilerParams(needs_layout_passes=False),
    )(Wf, se, nl_flat, a2hi)


@jax.jit
def kernel(node_embs, edge_embs, edge_list, node_list, W1, W2, a1, a2):
    el_flat = edge_list.astype(jnp.int32).reshape(M * A)
    nl_flat = node_list.astype(jnp.int32).reshape(N * L)
    a2c, a2lo = a2[:D, :], a2[D:, :]

    Wh, sn2, a2hi2 = _tc_stage1(node_embs, W1, a1, a2lo)
    new_edge = _sc_edge_agg(Wh, sn2.reshape(N), el_flat)
    Wf, se2, edge_out = _tc_stage3(new_edge, W2, a2c)
    node_out = _sc_node_agg(Wf, se2.reshape(M), nl_flat, a2hi2.reshape(N))
    return node_out, edge_out


# speedup vs baseline: 3.6183x; 1.1879x over previous
import functools

import jax
import jax.numpy as jnp
from jax import lax
from jax.experimental import pallas as pl
from jax.experimental.pallas import tpu as pltpu
from jax.experimental.pallas import tpu_sc as plsc

N = 8192
M = 16384
A = 8
L = 16
D = 256
ALPHA = 0.2
NEG = -9e15

NC, NS = 2, 16
NW = NC * NS
LANES = 16


def _tc1_body(x_ref, w1_ref, a1_ref, a2lo_ref, wh_ref, sn_ref, a2hi_ref):
    x = x_ref[...]
    wh = jnp.dot(x, w1_ref[...], preferred_element_type=jnp.float32)
    wh_ref[...] = wh
    lrelu = jnp.where(wh > 0, wh, ALPHA * wh)
    sn_ref[...] = jnp.dot(lrelu, a1_ref[...], preferred_element_type=jnp.float32)
    a2hi_ref[...] = jnp.dot(x, a2lo_ref[...], preferred_element_type=jnp.float32)


def _tc_stage1(node_embs, W1, a1, a2lo):
    BLK = 1024
    return pl.pallas_call(
        _tc1_body,
        grid=(N // BLK,),
        in_specs=[
            pl.BlockSpec((BLK, D), lambda i: (i, 0)),
            pl.BlockSpec((D, D), lambda i: (0, 0)),
            pl.BlockSpec((D, 1), lambda i: (0, 0)),
            pl.BlockSpec((D, 1), lambda i: (0, 0)),
        ],
        out_specs=[
            pl.BlockSpec((BLK, D), lambda i: (i, 0)),
            pl.BlockSpec((BLK, 1), lambda i: (i, 0)),
            pl.BlockSpec((BLK, 1), lambda i: (i, 0)),
        ],
        out_shape=[
            jax.ShapeDtypeStruct((N, D), jnp.float32),
            jax.ShapeDtypeStruct((N, 1), jnp.float32),
            jax.ShapeDtypeStruct((N, 1), jnp.float32),
        ],
    )(node_embs, W1, a1, a2lo)


def _tc3_body(x_ref, w2_ref, a2c_ref, wf_ref, se_ref, eo_ref):
    x = x_ref[...]
    wf = jnp.dot(x, w2_ref[...], preferred_element_type=jnp.float32)
    wf_ref[...] = wf
    se_ref[...] = jnp.dot(wf, a2c_ref[...], preferred_element_type=jnp.float32)
    eo_ref[...] = jnp.where(x > 0, x, jnp.exp(x) - 1.0)


def _tc_stage3(new_edge, W2, a2c):
    BLK = 1024
    return pl.pallas_call(
        _tc3_body,
        grid=(M // BLK,),
        in_specs=[
            pl.BlockSpec((BLK, D), lambda i: (i, 0)),
            pl.BlockSpec((D, D), lambda i: (0, 0)),
            pl.BlockSpec((D, 1), lambda i: (0, 0)),
        ],
        out_specs=[
            pl.BlockSpec((BLK, D), lambda i: (i, 0)),
            pl.BlockSpec((BLK, 1), lambda i: (i, 0)),
            pl.BlockSpec((BLK, D), lambda i: (i, 0)),
        ],
        out_shape=[
            jax.ShapeDtypeStruct((M, D), jnp.float32),
            jax.ShapeDtypeStruct((M, 1), jnp.float32),
            jax.ShapeDtypeStruct((M, D), jnp.float32),
        ],
    )(new_edge, W2, a2c)


_MESH = plsc.VectorSubcoreMesh(
    core_axis_name="c", subcore_axis_name="s", num_cores=NC, num_subcores=NS)

E_CH = 16
E_PW = M // NW
N_CH = 8
N_PW = N // NW


def _softmax_weights(scores):
    mx = scores[0]
    for s in scores[1:]:
        mx = jnp.maximum(mx, s)
    es = [jnp.exp(s - mx) for s in scores]
    tot = es[0]
    for e in es[1:]:
        tot = tot + e
    inv = 1.0 / tot
    return [e * inv for e in es]


def _edge_agg_body(wh_hbm, sn_hbm, el_hbm, out_hbm,
                   sn_l, idx0, idx1, rows0, rows1, out_l, w_buf, sem0, sem1):
    wid = lax.axis_index("s") * NC + lax.axis_index("c")
    pltpu.sync_copy(sn_hbm, sn_l)
    iota = lax.iota(jnp.int32, LANES)
    nchunk = E_PW // E_CH
    base0 = wid * E_PW

    def fetch(c, idx_l, rows_l, sem):
        pltpu.sync_copy(el_hbm.at[pl.ds((base0 + c * E_CH) * A, E_CH * A)],
                        idx_l)
        pltpu.async_copy(wh_hbm.at[idx_l], rows_l, sem)

    def process(c, idx_l, rows_l, sem):
        scores = []
        for a in range(A):
            nid = plsc.load_gather(idx_l, [iota * A + a])
            s = plsc.load_gather(sn_l, [nid])
            scores.append(jnp.where(s > 0, s, NEG))
        ws = _softmax_weights(scores)
        for a in range(A):
            w_buf[pl.ds(a * LANES, LANES)] = ws[a]
        pltpu.make_async_copy(wh_hbm.at[idx_l], rows_l, sem).wait()

        def edge(e, _):
            ev = jnp.full((LANES,), e, dtype=jnp.int32)
            wv = [plsc.load_gather(w_buf, [ev + (a * LANES)]) for a in range(A)]
            rb = e * A
            for dc in range(D // LANES):
                sl = pl.ds(dc * LANES, LANES)
                acc = wv[0] * rows_l[rb, sl]
                for a in range(1, A):
                    acc = acc + wv[a] * rows_l[rb + a, sl]
                out_l[e, sl] = acc
            return 0

        lax.fori_loop(0, E_CH, edge, 0)
        pltpu.sync_copy(out_l, out_hbm.at[pl.ds(base0 + c * E_CH, E_CH)])

    fetch(0, idx0, rows0, sem0)

    def body(g, _):
        c0 = 2 * g
        fetch(c0 + 1, idx1, rows1, sem1)
        process(c0, idx0, rows0, sem0)

        @pl.when(c0 + 2 < nchunk)
        def _():
            fetch(c0 + 2, idx0, rows0, sem0)

        process(c0 + 1, idx1, rows1, sem1)
        return 0

    lax.fori_loop(0, nchunk // 2, body, 0)


def _sc_edge_agg(Wh, sn, el_flat):
    return pl.kernel(
        _edge_agg_body,
        out_type=jax.ShapeDtypeStruct((M, D), jnp.float32),
        mesh=_MESH,
        scratch_types=[
            pltpu.VMEM((N,), jnp.float32),
            pltpu.VMEM((E_CH * A,), jnp.int32),
            pltpu.VMEM((E_CH * A,), jnp.int32),
            pltpu.VMEM((E_CH * A, D), jnp.float32),
            pltpu.VMEM((E_CH * A, D), jnp.float32),
            pltpu.VMEM((E_CH, D), jnp.float32),
            pltpu.VMEM((A * LANES,), jnp.float32),
            pltpu.SemaphoreType.DMA,
            pltpu.SemaphoreType.DMA,
        ],
        compiler_params=pltpu.CompilerParams(needs_layout_passes=False),
    )(Wh, sn, el_flat)


def _node_agg_body(wf_hbm, se_hbm, nl_hbm, a2hi_hbm, out_hbm,
                   se_l, idx0, idx1, rows0, rows1, out_l, w_buf,
                   a2h0, a2h1, sem0, sem1):
    wid = lax.axis_index("s") * NC + lax.axis_index("c")
    pltpu.sync_copy(se_hbm, se_l)
    iota = lax.iota(jnp.int32, LANES)
    lane_ok = iota < N_CH
    nchunk = N_PW // N_CH
    base0 = wid * N_PW

    def fetch(c, idx_l, a2h_l, rows_l, sem):
        pltpu.sync_copy(nl_hbm.at[pl.ds((base0 + c * N_CH) * L, N_CH * L)],
                        idx_l)
        pltpu.sync_copy(a2hi_hbm.at[pl.ds(base0 + c * N_CH, N_CH)], a2h_l)
        pltpu.async_copy(wf_hbm.at[idx_l], rows_l, sem)

    def process(c, idx_l, a2h_l, rows_l, sem):
        a2h = plsc.load_gather(a2h_l, [iota], mask=lane_ok)
        scores = []
        for l in range(L):
            eid = plsc.load_gather(idx_l, [iota * L + l], mask=lane_ok)
            s = plsc.load_gather(se_l, [eid], mask=lane_ok)
            scores.append(jnp.where(s != 0, s + a2h, s))
        ws = _softmax_weights(scores)
        for l in range(L):
            w_buf[pl.ds(l * LANES, LANES)] = ws[l]
        pltpu.make_async_copy(wf_hbm.at[idx_l], rows_l, sem).wait()

        def node(n, _):
            nv = jnp.full((LANES,), n, dtype=jnp.int32)
            wv = [plsc.load_gather(w_buf, [nv + (l * LANES)]) for l in range(L)]
            rb = n * L
            for dc in range(D // LANES):
                sl = pl.ds(dc * LANES, LANES)
                acc = wv[0] * rows_l[rb, sl]
                for l in range(1, L):
                    acc = acc + wv[l] * rows_l[rb + l, sl]
                out_l[n, sl] = jnp.where(acc > 0, acc, jnp.exp(acc) - 1.0)
            return 0

        lax.fori_loop(0, N_CH, node, 0)
        pltpu.sync_copy(out_l, out_hbm.at[pl.ds(base0 + c * N_CH, N_CH)])

    fetch(0, idx0, a2h0, rows0, sem0)

    def body(g, _):
        c0 = 2 * g
        fetch(c0 + 1, idx1, a2h1, rows1, sem1)
        process(c0, idx0, a2h0, rows0, sem0)

        @pl.when(c0 + 2 < nchunk)
        def _():
            fetch(c0 + 2, idx0, a2h0, rows0, sem0)

        process(c0 + 1, idx1, a2h1, rows1, sem1)
        return 0

    lax.fori_loop(0, nchunk // 2, body, 0)


def _sc_node_agg(Wf, se, nl_flat, a2hi):
    return pl.kernel(
        _node_agg_body,
        out_type=jax.ShapeDtypeStruct((N, D), jnp.float32),
        mesh=_MESH,
        scratch_types=[
            pltpu.VMEM((M,), jnp.float32),
            pltpu.VMEM((N_CH * L,), jnp.int32),
            pltpu.VMEM((N_CH * L,), jnp.int32),
            pltpu.VMEM((N_CH * L, D), jnp.float32),
            pltpu.VMEM((N_CH * L, D), jnp.float32),
            pltpu.VMEM((N_CH, D), jnp.float32),
            pltpu.VMEM((L * LANES,), jnp.float32),
            pltpu.VMEM((N_CH,), jnp.float32),
            pltpu.VMEM((N_CH,), jnp.float32),
            pltpu.SemaphoreType.DMA,
            pltpu.SemaphoreType.DMA,
        ],
        compiler_params=pltpu.CompilerParams(needs_layout_passes=False),
    )(Wf, se, nl_flat, a2hi)


@jax.jit
def kernel(node_embs, edge_embs, edge_list, node_list, W1, W2, a1, a2):
    el_flat = edge_list.astype(jnp.int32).reshape(M * A)
    nl_flat = node_list.astype(jnp.int32).reshape(N * L)
    a2c, a2lo = a2[:D, :], a2[D:, :]

    Wh, sn2, a2hi2 = _tc_stage1(node_embs, W1, a1, a2lo)
    new_edge = _sc_edge_agg(Wh, sn2.reshape(N), el_flat)
    Wf, se2, edge_out = _tc_stage3(new_edge, W2, a2c)
    node_out = _sc_node_agg(Wf, se2.reshape(M), nl_flat, a2hi2.reshape(N))
    return node_out, edge_out
